# Initial kernel scaffold; baseline (speedup 1.0000x reference)
import jax
import jax.numpy as jnp
from jax.experimental import pallas as pl


def kernel(x, weight):
    raise NotImplementedError("write your pallas kernel here")


# speedup vs baseline: 2.2705x; 2.2705x over previous
import functools

import jax
import jax.numpy as jnp
from jax.experimental import pallas as pl

_BN = 256


def _vq_body(x_ref, wT_ref, w_ref, qd_ref, qi_ref):
    xb = x_ref[...]
    wT = wT_ref[...]
    q = wT.shape[1]
    bn = xb.shape[0]

    acc = None
    for g in range(4):
        terms = []
        for s in range(8):
            j = 8 * g + s
            d = wT[j, :][None, :] - xb[:, j][:, None]
            terms.append(d * d)
        b = [terms[s] + terms[s + 4] for s in range(4)]
        c0 = b[0] + b[2]
        c1 = b[1] + b[3]
        e = c0 + c1
        acc = e if acc is None else acc + e

    m = jnp.min(acc, axis=1, keepdims=True)
    iota = jax.lax.broadcasted_iota(jnp.int32, (bn, q), 1)
    idx = jnp.min(jnp.where(acc == m, iota, q), axis=1)
    qi_ref[...] = idx[:, None]

    onehot = (iota == idx[:, None]).astype(jnp.float32)
    qd_ref[...] = jax.lax.dot_general(
        onehot, w_ref[...], (((1,), (0,)), ((), ())),
        precision=jax.lax.Precision.HIGHEST)


@functools.partial(jax.jit, static_argnames=())
def kernel(x, weight):
    input_shape = x.shape
    e = weight.shape[1]
    q = weight.shape[0]
    xf = x.reshape(-1, e)
    n = xf.shape[0]
    wT = weight.T

    q_data, q_idx = pl.pallas_call(
        _vq_body,
        grid=(n // _BN,),
        in_specs=[
            pl.BlockSpec((_BN, e), lambda i: (i, 0)),
            pl.BlockSpec((e, q), lambda i: (0, 0)),
            pl.BlockSpec((q, e), lambda i: (0, 0)),
        ],
        out_specs=[
            pl.BlockSpec((_BN, e), lambda i: (i, 0)),
            pl.BlockSpec((_BN, 1), lambda i: (i, 0)),
        ],
        out_shape=[
            jax.ShapeDtypeStruct((n, e), jnp.float32),
            jax.ShapeDtypeStruct((n, 1), jnp.int32),
        ],
    )(xf, wT, weight)
    return (q_data.reshape(input_shape),
            q_idx.reshape(input_shape[:-1]))


# speedup vs baseline: 2.4436x; 1.0762x over previous
import functools

import jax
import jax.numpy as jnp
from jax.experimental import pallas as pl

_BN = 256


def _vq_body(x_ref, wT_ref, w_hi_ref, w_mid_ref, w_lo_ref, qd_ref, qi_ref):
    xb = x_ref[...]
    wT = wT_ref[...]
    q = wT.shape[1]
    bn = xb.shape[0]

    acc = None
    for g in range(4):
        terms = []
        for s in range(8):
            j = 8 * g + s
            d = wT[j, :][None, :] - xb[:, j][:, None]
            terms.append(d * d)
        b = [terms[s] + terms[s + 4] for s in range(4)]
        c0 = b[0] + b[2]
        c1 = b[1] + b[3]
        e = c0 + c1
        acc = e if acc is None else acc + e

    m = jnp.min(acc, axis=1, keepdims=True)
    iota = jax.lax.broadcasted_iota(jnp.int32, (bn, q), 1)
    idx = jnp.min(jnp.where(acc == m, iota, q), axis=1)
    qi_ref[...] = idx[:, None]

    onehot = (iota == idx[:, None]).astype(jnp.bfloat16)
    dims = (((1,), (0,)), ((), ()))
    g_hi = jax.lax.dot_general(onehot, w_hi_ref[...], dims,
                               preferred_element_type=jnp.float32)
    g_mid = jax.lax.dot_general(onehot, w_mid_ref[...], dims,
                                preferred_element_type=jnp.float32)
    g_lo = jax.lax.dot_general(onehot, w_lo_ref[...], dims,
                               preferred_element_type=jnp.float32)
    qd_ref[...] = (g_hi + g_mid) + g_lo


@functools.partial(jax.jit, static_argnames=())
def kernel(x, weight):
    input_shape = x.shape
    e = weight.shape[1]
    q = weight.shape[0]
    xf = x.reshape(-1, e)
    n = xf.shape[0]
    wT = weight.T
    w_hi = weight.astype(jnp.bfloat16)
    r1 = weight - w_hi.astype(jnp.float32)
    w_mid = r1.astype(jnp.bfloat16)
    w_lo = (r1 - w_mid.astype(jnp.float32)).astype(jnp.bfloat16)

    q_data, q_idx = pl.pallas_call(
        _vq_body,
        grid=(n // _BN,),
        in_specs=[
            pl.BlockSpec((_BN, e), lambda i: (i, 0)),
            pl.BlockSpec((e, q), lambda i: (0, 0)),
            pl.BlockSpec((q, e), lambda i: (0, 0)),
            pl.BlockSpec((q, e), lambda i: (0, 0)),
            pl.BlockSpec((q, e), lambda i: (0, 0)),
        ],
        out_specs=[
            pl.BlockSpec((_BN, e), lambda i: (i, 0)),
            pl.BlockSpec((_BN, 1), lambda i: (i, 0)),
        ],
        out_shape=[
            jax.ShapeDtypeStruct((n, e), jnp.float32),
            jax.ShapeDtypeStruct((n, 1), jnp.int32),
        ],
    )(xf, wT, w_hi, w_mid, w_lo)
    return (q_data.reshape(input_shape),
            q_idx.reshape(input_shape[:-1]))
